# Initial kernel scaffold; baseline (speedup 1.0000x reference)
import jax
import jax.numpy as jnp
from jax.experimental import pallas as pl


def kernel(x, edge_index, batch, num_graphs, ptr, W1, b1, W2, b2, Wf1, bf1, Wf2, bf2):
    raise NotImplementedError("write your pallas kernel here")


# speedup vs baseline: 14.9151x; 14.9151x over previous
import functools

import jax
import jax.numpy as jnp
from jax import lax
from jax.experimental import pallas as pl
from jax.experimental.pallas import tpu as pltpu
from jax.experimental.pallas import tpu_sc as plsc

N = 10000
E = 160000
D_IN = 256
D_H = 512
G = 16

LC = 128
NCH = D_H // LC
NPAIR = NCH // 2
NCORE = 2
NTILE = 16
CPC = NCH // NCORE

EG = 128
GPT = 79
EPT = GPT * EG
E_PAD = NTILE * EPT

NP = 10112
ROWS_PT = NP // NTILE

HIST_W = 16
NH = 10240
HROWS_PT = NH // NTILE

BR = 1000
NR = N // BR

_mesh = plsc.VectorSubcoreMesh(core_axis_name="c", subcore_axis_name="s")


@functools.partial(
    pl.kernel,
    mesh=_mesh,
    out_type=jax.ShapeDtypeStruct((NH, HIST_W), jnp.float32),
    scratch_types=[
        pltpu.VMEM((GPT, EG), jnp.int32),
        pltpu.VMEM((EG, HIST_W), jnp.float32),
        pltpu.VMEM_SHARED((NH, HIST_W), jnp.float32),
    ],
)
def _deg_kernel(dst_hbm, deg_hbm, dstv, ones_rows, hist_sh):
    c = lax.axis_index("c")
    s = lax.axis_index("s")
    pltpu.sync_copy(dst_hbm.at[s], dstv)
    lane = lax.iota(jnp.int32, 16)
    one_hot = jnp.where(lane == 0, 1.0, 0.0).astype(jnp.float32)
    zero = jnp.zeros((16,), jnp.float32)

    def _fillz(i, _):
        ones_rows[i] = zero
        return 0

    lax.fori_loop(0, EG, _fillz, 0)
    for z in range(HROWS_PT // EG):
        pltpu.sync_copy(ones_rows, hist_sh.at[pl.ds(s * HROWS_PT + z * EG, EG)])
    plsc.subcore_barrier()

    def _fill1(i, _):
        ones_rows[i] = one_hot
        return 0

    lax.fori_loop(0, EG, _fill1, 0)

    def _acc(g, _):
        pltpu.sync_copy(ones_rows, hist_sh.at[dstv.at[g]], add=True)
        return 0

    lax.fori_loop(0, GPT, _acc, 0)
    plsc.subcore_barrier()
    base = (NH // NCORE) * c + (NH // NCORE // NTILE) * s
    span = NH // NCORE // NTILE
    pltpu.sync_copy(hist_sh.at[pl.ds(base, span)], deg_hbm.at[pl.ds(base, span)])


@functools.partial(
    pl.kernel,
    mesh=_mesh,
    out_type=jax.ShapeDtypeStruct((NCH, NP, LC), jnp.float32),
    scratch_types=[
        pltpu.VMEM((GPT, EG), jnp.int32),
        pltpu.VMEM((1, EG), jnp.int32),
        pltpu.VMEM((1, EG), jnp.int32),
        pltpu.VMEM((EG, LC), jnp.float32),
        pltpu.VMEM((EG, LC), jnp.float32),
        pltpu.VMEM_SHARED((NP, LC), jnp.float32),
        pltpu.SemaphoreType.DMA,
        pltpu.SemaphoreType.DMA,
        pltpu.SemaphoreType.DMA,
        pltpu.SemaphoreType.DMA,
    ],
)
def _edge_scatter_kernel(h_hbm, src_hbm, dst_hbm, acc_hbm,
                         srcv, dst0, dst1, rows0, rows1, accsh,
                         sem0, sem1, semd0, semd1):
    c = lax.axis_index("c")
    s = lax.axis_index("s")
    pltpu.sync_copy(src_hbm.at[s], srcv)
    dsts = dst_hbm.at[s]
    zero = jnp.zeros((16,), jnp.float32)

    for ci in range(CPC):
        ch = c * CPC + ci
        hsrc = h_hbm.at[ch]

        def _zb(i, _):
            for k in range(LC // 16):
                rows0[i, pl.ds(k * 16, 16)] = zero
            return 0

        lax.fori_loop(0, EG, _zb, 0)
        nfull = ROWS_PT // EG
        for z in range(nfull):
            pltpu.sync_copy(rows0, accsh.at[pl.ds(s * ROWS_PT + z * EG, EG)])
        rem = ROWS_PT - nfull * EG
        if rem:
            pltpu.sync_copy(rows0.at[pl.ds(0, rem)],
                            accsh.at[pl.ds(s * ROWS_PT + nfull * EG, rem)])
        plsc.subcore_barrier()

        def _gather(g, buf, sem):
            return pltpu.make_async_copy(hsrc.at[srcv.at[g]], buf, sem)

        def _dstld(g, buf, sem):
            return pltpu.make_async_copy(dsts.at[pl.ds(g, 1)], buf, sem)

        _gather(0, rows0, sem0).start()
        _dstld(0, dst0, semd0).start()

        def _body(p, _):
            g0 = 2 * p
            g1 = g0 + 1
            _gather(g1, rows1, sem1).start()
            _dstld(g1, dst1, semd1).start()
            _gather(g0, rows0, sem0).wait()
            _dstld(g0, dst0, semd0).wait()
            pltpu.sync_copy(rows0, accsh.at[dst0.at[0]], add=True)
            _gather(g0 + 2, rows0, sem0).start()
            _dstld(g0 + 2, dst0, semd0).start()
            _gather(g1, rows1, sem1).wait()
            _dstld(g1, dst1, semd1).wait()
            pltpu.sync_copy(rows1, accsh.at[dst1.at[0]], add=True)
            return 0

        lax.fori_loop(0, (GPT - 1) // 2, _body, 0)
        _gather(GPT - 1, rows0, sem0).wait()
        _dstld(GPT - 1, dst0, semd0).wait()
        pltpu.sync_copy(rows0, accsh.at[dst0.at[0]], add=True)
        plsc.subcore_barrier()
        pltpu.sync_copy(accsh.at[pl.ds(s * ROWS_PT, ROWS_PT)],
                        acc_hbm.at[ch].at[pl.ds(s * ROWS_PT, ROWS_PT)])


def _mm1_body(x_ref, w_ref, deg_ref, out_ref):
    h = jnp.dot(x_ref[...], w_ref[...], preferred_element_type=jnp.float32)
    dinv = lax.rsqrt(deg_ref[...] + 1.0)
    hs = h * dinv
    out_ref[0] = hs[:, :LC]
    out_ref[1] = hs[:, LC:]


def _mm1(x, W1, degf):
    return pl.pallas_call(
        _mm1_body,
        grid=(NR, NPAIR),
        in_specs=[
            pl.BlockSpec((BR, D_IN), lambda i, j: (i, 0)),
            pl.BlockSpec((D_IN, 2 * LC), lambda i, j: (0, j)),
            pl.BlockSpec((BR, 1), lambda i, j: (i, 0)),
        ],
        out_specs=pl.BlockSpec((2, BR, LC), lambda i, j: (j, i, 0)),
        out_shape=jax.ShapeDtypeStruct((NCH, N, LC), jnp.float32),
    )(x, W1, degf)


def _mm2_body(acc_ref, h_ref, deg_ref, b_ref, w_ref, out_ref, accum):
    k = pl.program_id(2)
    dinv = lax.rsqrt(deg_ref[...] + 1.0)
    acc = jnp.concatenate([acc_ref[0], acc_ref[1]], axis=1)
    h = jnp.concatenate([h_ref[0], h_ref[1]], axis=1)
    y = jnp.maximum((acc + h) * dinv + b_ref[0], 0.0)
    part = jnp.dot(y, w_ref[...], preferred_element_type=jnp.float32)

    @pl.when(k == 0)
    def _():
        accum[...] = jnp.zeros_like(accum)

    accum[...] += part

    @pl.when(k == NPAIR - 1)
    def _():
        h2 = accum[...] * dinv
        out_ref[0] = h2[:, :LC]
        out_ref[1] = h2[:, LC:]


def _mm2(acc1, h1s, degf, b1r, W2):
    return pl.pallas_call(
        _mm2_body,
        grid=(NR, NPAIR, NPAIR),
        in_specs=[
            pl.BlockSpec((2, BR, LC), lambda i, j, k: (k, i, 0)),
            pl.BlockSpec((2, BR, LC), lambda i, j, k: (k, i, 0)),
            pl.BlockSpec((BR, 1), lambda i, j, k: (i, 0)),
            pl.BlockSpec((1, 1, 2 * LC), lambda i, j, k: (k, 0, 0)),
            pl.BlockSpec((2 * LC, 2 * LC), lambda i, j, k: (k, j)),
        ],
        out_specs=pl.BlockSpec((2, BR, LC), lambda i, j, k: (j, i, 0)),
        out_shape=jax.ShapeDtypeStruct((NCH, N, LC), jnp.float32),
        scratch_shapes=[pltpu.VMEM((BR, 2 * LC), jnp.float32)],
    )(acc1, h1s, degf, b1r, W2)


def _pool_body(acc_ref, h_ref, deg_ref, b_ref, batch_ref, out_ref):
    i = pl.program_id(1)
    dinv = lax.rsqrt(deg_ref[...] + 1.0)
    acc = jnp.concatenate([acc_ref[0], acc_ref[1]], axis=1)
    h = jnp.concatenate([h_ref[0], h_ref[1]], axis=1)
    y = jnp.maximum((acc + h) * dinv + b_ref[0], 0.0)
    bidx = batch_ref[...]
    neg = jnp.float32(-jnp.inf)

    @pl.when(i == 0)
    def _():
        out_ref[...] = jnp.full((G, 2 * LC), neg, jnp.float32)

    rows = [jnp.max(jnp.where(bidx == g, y, neg), axis=0, keepdims=True)
            for g in range(G)]
    out_ref[...] = jnp.maximum(out_ref[...], jnp.concatenate(rows, axis=0))


def _pool(acc2, h2s, degf, b2r, batch2d):
    return pl.pallas_call(
        _pool_body,
        grid=(NPAIR, NR),
        in_specs=[
            pl.BlockSpec((2, BR, LC), lambda j, i: (j, i, 0)),
            pl.BlockSpec((2, BR, LC), lambda j, i: (j, i, 0)),
            pl.BlockSpec((BR, 1), lambda j, i: (i, 0)),
            pl.BlockSpec((1, 1, 2 * LC), lambda j, i: (j, 0, 0)),
            pl.BlockSpec((BR, 1), lambda j, i: (i, 0)),
        ],
        out_specs=pl.BlockSpec((G, 2 * LC), lambda j, i: (0, j)),
        out_shape=jax.ShapeDtypeStruct((G, D_H), jnp.float32),
    )(acc2, h2s, degf, b2r, batch2d)


def _mlp_body(p_ref, w1_ref, b1_ref, w2_ref, b2_ref, out_ref):
    hid = jnp.maximum(
        jnp.dot(p_ref[...], w1_ref[...], preferred_element_type=jnp.float32)
        + b1_ref[...], 0.0)
    out_ref[...] = jnp.maximum(
        jnp.dot(hid, w2_ref[...], preferred_element_type=jnp.float32)
        + b2_ref[...], 0.0)


def _mlp(pooled, Wf1, bf1r, Wf2, bf2r):
    return pl.pallas_call(
        _mlp_body,
        out_shape=jax.ShapeDtypeStruct((G, 32), jnp.float32),
    )(pooled, Wf1, bf1r, Wf2, bf2r)


def kernel(x, edge_index, batch, num_graphs, ptr, W1, b1, W2, b2,
           Wf1, bf1, Wf2, bf2):
    pad = E_PAD - E
    pad_src = (jnp.arange(pad, dtype=jnp.int32) * 53) % N
    pad_dst = N + jnp.arange(pad, dtype=jnp.int32) % (NP - N)
    src2 = jnp.concatenate([edge_index[0], pad_src]).reshape(NTILE, GPT, EG)
    dst2 = jnp.concatenate([edge_index[1], pad_dst]).reshape(NTILE, GPT, EG)

    degh = _deg_kernel(dst2)
    degf = degh[:N, 0:1]

    h1s = _mm1(x, W1, degf)
    acc1 = _edge_scatter_kernel(h1s, src2, dst2)
    h2s = _mm2(acc1, h1s, degf, b1.reshape(NPAIR, 1, 2 * LC), W2)
    acc2 = _edge_scatter_kernel(h2s, src2, dst2)
    pooled = _pool(acc2, h2s, degf, b2.reshape(NPAIR, 1, 2 * LC),
                   batch.reshape(N, 1))
    return _mlp(pooled, Wf1, bf1.reshape(1, 64), Wf2, bf2.reshape(1, 32))


# speedup vs baseline: 15.4544x; 1.0362x over previous
import functools

import jax
import jax.numpy as jnp
from jax import lax
from jax.experimental import pallas as pl
from jax.experimental.pallas import tpu as pltpu
from jax.experimental.pallas import tpu_sc as plsc

N = 10000
E = 160000
D_IN = 256
D_H = 512
G = 16

LC = 128
NCH = D_H // LC
NPAIR = NCH // 2
NCORE = 2
NTILE = 16
CPC = NCH // NCORE

EG = 128
GPT = 79
EPT = GPT * EG
E_PAD = NTILE * EPT

NP = 10112
ROWS_PT = NP // NTILE

HIST_W = 16
NH = 10240
HROWS_PT = NH // NTILE

BR = 2000
NR = N // BR

_mesh = plsc.VectorSubcoreMesh(core_axis_name="c", subcore_axis_name="s")


@functools.partial(
    pl.kernel,
    mesh=_mesh,
    out_type=jax.ShapeDtypeStruct((NH, HIST_W), jnp.float32),
    scratch_types=[
        pltpu.VMEM((GPT, EG), jnp.int32),
        pltpu.VMEM((EG, HIST_W), jnp.float32),
        pltpu.VMEM_SHARED((NH, HIST_W), jnp.float32),
    ],
)
def _deg_kernel(dst_hbm, deg_hbm, dstv, ones_rows, hist_sh):
    c = lax.axis_index("c")
    s = lax.axis_index("s")
    pltpu.sync_copy(dst_hbm.at[s], dstv)
    lane = lax.iota(jnp.int32, 16)
    one_hot = jnp.where(lane == 0, 1.0, 0.0).astype(jnp.float32)
    zero = jnp.zeros((16,), jnp.float32)

    def _fillz(i, _):
        ones_rows[i] = zero
        return 0

    lax.fori_loop(0, EG, _fillz, 0)
    for z in range(HROWS_PT // EG):
        pltpu.sync_copy(ones_rows, hist_sh.at[pl.ds(s * HROWS_PT + z * EG, EG)])
    plsc.subcore_barrier()

    def _fill1(i, _):
        ones_rows[i] = one_hot
        return 0

    lax.fori_loop(0, EG, _fill1, 0)

    def _acc(g, _):
        pltpu.sync_copy(ones_rows, hist_sh.at[dstv.at[g]], add=True)
        return 0

    lax.fori_loop(0, GPT, _acc, 0)
    plsc.subcore_barrier()
    base = (NH // NCORE) * c + (NH // NCORE // NTILE) * s
    span = NH // NCORE // NTILE
    pltpu.sync_copy(hist_sh.at[pl.ds(base, span)], deg_hbm.at[pl.ds(base, span)])


@functools.partial(
    pl.kernel,
    mesh=_mesh,
    out_type=jax.ShapeDtypeStruct((NCH, NP, LC), jnp.float32),
    scratch_types=[
        pltpu.VMEM((GPT, EG), jnp.int32),
        pltpu.VMEM((1, EG), jnp.int32),
        pltpu.VMEM((1, EG), jnp.int32),
        pltpu.VMEM((EG, LC), jnp.float32),
        pltpu.VMEM((EG, LC), jnp.float32),
        pltpu.VMEM_SHARED((NP, LC), jnp.float32),
        pltpu.SemaphoreType.DMA,
        pltpu.SemaphoreType.DMA,
        pltpu.SemaphoreType.DMA,
        pltpu.SemaphoreType.DMA,
    ],
)
def _edge_scatter_kernel(h_hbm, src_hbm, dst_hbm, acc_hbm,
                         srcv, dst0, dst1, rows0, rows1, accsh,
                         sem0, sem1, semd0, semd1):
    c = lax.axis_index("c")
    s = lax.axis_index("s")
    pltpu.sync_copy(src_hbm.at[s], srcv)
    dsts = dst_hbm.at[s]
    zero = jnp.zeros((16,), jnp.float32)

    for ci in range(CPC):
        ch = c * CPC + ci
        hsrc = h_hbm.at[ch]

        def _zb(i, _):
            for k in range(LC // 16):
                rows0[i, pl.ds(k * 16, 16)] = zero
            return 0

        lax.fori_loop(0, EG, _zb, 0)
        nfull = ROWS_PT // EG
        for z in range(nfull):
            pltpu.sync_copy(rows0, accsh.at[pl.ds(s * ROWS_PT + z * EG, EG)])
        rem = ROWS_PT - nfull * EG
        if rem:
            pltpu.sync_copy(rows0.at[pl.ds(0, rem)],
                            accsh.at[pl.ds(s * ROWS_PT + nfull * EG, rem)])
        plsc.subcore_barrier()

        def _gather(g, buf, sem):
            return pltpu.make_async_copy(hsrc.at[srcv.at[g]], buf, sem)

        def _dstld(g, buf, sem):
            return pltpu.make_async_copy(dsts.at[pl.ds(g, 1)], buf, sem)

        _gather(0, rows0, sem0).start()
        _dstld(0, dst0, semd0).start()

        def _body(p, _):
            g0 = 2 * p
            g1 = g0 + 1
            _gather(g1, rows1, sem1).start()
            _dstld(g1, dst1, semd1).start()
            _gather(g0, rows0, sem0).wait()
            _dstld(g0, dst0, semd0).wait()
            pltpu.sync_copy(rows0, accsh.at[dst0.at[0]], add=True)
            _gather(g0 + 2, rows0, sem0).start()
            _dstld(g0 + 2, dst0, semd0).start()
            _gather(g1, rows1, sem1).wait()
            _dstld(g1, dst1, semd1).wait()
            pltpu.sync_copy(rows1, accsh.at[dst1.at[0]], add=True)
            return 0

        lax.fori_loop(0, (GPT - 1) // 2, _body, 0)
        _gather(GPT - 1, rows0, sem0).wait()
        _dstld(GPT - 1, dst0, semd0).wait()
        pltpu.sync_copy(rows0, accsh.at[dst0.at[0]], add=True)
        plsc.subcore_barrier()
        pltpu.sync_copy(accsh.at[pl.ds(s * ROWS_PT, ROWS_PT)],
                        acc_hbm.at[ch].at[pl.ds(s * ROWS_PT, ROWS_PT)])


def _mm1_body(x_ref, w_ref, deg_ref, out_ref):
    h = jnp.dot(x_ref[...], w_ref[...], preferred_element_type=jnp.float32)

    dinv = lax.rsqrt(deg_ref[...] + 1.0)
    hs = h * dinv
    out_ref[0] = hs[:, :LC]
    out_ref[1] = hs[:, LC:]


def _mm1(x, W1, degf):
    return pl.pallas_call(
        _mm1_body,
        grid=(NR, NPAIR),
        in_specs=[
            pl.BlockSpec((BR, D_IN), lambda i, j: (i, 0)),
            pl.BlockSpec((D_IN, 2 * LC), lambda i, j: (0, j)),
            pl.BlockSpec((BR, 1), lambda i, j: (i, 0)),
        ],
        out_specs=pl.BlockSpec((2, BR, LC), lambda i, j: (j, i, 0)),
        out_shape=jax.ShapeDtypeStruct((NCH, N, LC), jnp.float32),
    )(x, W1, degf)


def _mm2_body(acc_ref, h_ref, deg_ref, b_ref, w_ref, out_ref, accum):
    k = pl.program_id(2)
    dinv = lax.rsqrt(deg_ref[...] + 1.0)
    acc = jnp.concatenate([acc_ref[0], acc_ref[1]], axis=1)
    h = jnp.concatenate([h_ref[0], h_ref[1]], axis=1)
    y = jnp.maximum((acc + h) * dinv + b_ref[0], 0.0)
    part = jnp.dot(y, w_ref[...], preferred_element_type=jnp.float32)

    @pl.when(k == 0)
    def _():
        accum[...] = jnp.zeros_like(accum)

    accum[...] += part

    @pl.when(k == NPAIR - 1)
    def _():
        h2 = accum[...] * dinv
        out_ref[0] = h2[:, :LC]
        out_ref[1] = h2[:, LC:]


def _mm2(acc1, h1s, degf, b1r, W2):
    return pl.pallas_call(
        _mm2_body,
        grid=(NR, NPAIR, NPAIR),
        in_specs=[
            pl.BlockSpec((2, BR, LC), lambda i, j, k: (k, i, 0)),
            pl.BlockSpec((2, BR, LC), lambda i, j, k: (k, i, 0)),
            pl.BlockSpec((BR, 1), lambda i, j, k: (i, 0)),
            pl.BlockSpec((1, 1, 2 * LC), lambda i, j, k: (k, 0, 0)),
            pl.BlockSpec((2 * LC, 2 * LC), lambda i, j, k: (k, j)),
        ],
        out_specs=pl.BlockSpec((2, BR, LC), lambda i, j, k: (j, i, 0)),
        out_shape=jax.ShapeDtypeStruct((NCH, N, LC), jnp.float32),
        scratch_shapes=[pltpu.VMEM((BR, 2 * LC), jnp.float32)],
    )(acc1, h1s, degf, b1r, W2)


def _pool_body(acc_ref, h_ref, deg_ref, b_ref, batch_ref, out_ref):
    i = pl.program_id(1)
    dinv = lax.rsqrt(deg_ref[...] + 1.0)
    acc = jnp.concatenate([acc_ref[0], acc_ref[1]], axis=1)
    h = jnp.concatenate([h_ref[0], h_ref[1]], axis=1)
    y = jnp.maximum((acc + h) * dinv + b_ref[0], 0.0)
    bidx = batch_ref[...]
    neg = jnp.float32(-jnp.inf)

    @pl.when(i == 0)
    def _():
        out_ref[...] = jnp.full((G, 2 * LC), neg, jnp.float32)

    rows = [jnp.max(jnp.where(bidx == g, y, neg), axis=0, keepdims=True)
            for g in range(G)]
    out_ref[...] = jnp.maximum(out_ref[...], jnp.concatenate(rows, axis=0))


def _pool(acc2, h2s, degf, b2r, batch2d):
    return pl.pallas_call(
        _pool_body,
        grid=(NPAIR, NR),
        in_specs=[
            pl.BlockSpec((2, BR, LC), lambda j, i: (j, i, 0)),
            pl.BlockSpec((2, BR, LC), lambda j, i: (j, i, 0)),
            pl.BlockSpec((BR, 1), lambda j, i: (i, 0)),
            pl.BlockSpec((1, 1, 2 * LC), lambda j, i: (j, 0, 0)),
            pl.BlockSpec((BR, 1), lambda j, i: (i, 0)),
        ],
        out_specs=pl.BlockSpec((G, 2 * LC), lambda j, i: (0, j)),
        out_shape=jax.ShapeDtypeStruct((G, D_H), jnp.float32),
    )(acc2, h2s, degf, b2r, batch2d)


def _mlp_body(p_ref, w1_ref, b1_ref, w2_ref, b2_ref, out_ref):
    hid = jnp.maximum(
        jnp.dot(p_ref[...], w1_ref[...], preferred_element_type=jnp.float32)
        + b1_ref[...], 0.0)
    out_ref[...] = jnp.maximum(
        jnp.dot(hid, w2_ref[...], preferred_element_type=jnp.float32)
        + b2_ref[...], 0.0)


def _mlp(pooled, Wf1, bf1r, Wf2, bf2r):
    return pl.pallas_call(
        _mlp_body,
        out_shape=jax.ShapeDtypeStruct((G, 32), jnp.float32),
    )(pooled, Wf1, bf1r, Wf2, bf2r)


def kernel(x, edge_index, batch, num_graphs, ptr, W1, b1, W2, b2,
           Wf1, bf1, Wf2, bf2):
    pad = E_PAD - E
    pad_src = (jnp.arange(pad, dtype=jnp.int32) * 53) % N
    pad_dst = N + jnp.arange(pad, dtype=jnp.int32) % (NP - N)
    src2 = jnp.concatenate([edge_index[0], pad_src]).reshape(NTILE, GPT, EG)
    dst2 = jnp.concatenate([edge_index[1], pad_dst]).reshape(NTILE, GPT, EG)

    degh = _deg_kernel(dst2)
    degf = degh[:N, 0:1]

    h1s = _mm1(x, W1, degf)
    acc1 = _edge_scatter_kernel(h1s, src2, dst2)
    h2s = _mm2(acc1, h1s, degf, b1.reshape(NPAIR, 1, 2 * LC), W2)
    acc2 = _edge_scatter_kernel(h2s, src2, dst2)
    pooled = _pool(acc2, h2s, degf, b2.reshape(NPAIR, 1, 2 * LC),
                   batch.reshape(N, 1))
    return _mlp(pooled, Wf1, bf1.reshape(1, 64), Wf2, bf2.reshape(1, 32))


# speedup vs baseline: 16.5860x; 1.0732x over previous
import functools

import jax
import jax.numpy as jnp
from jax import lax
from jax.experimental import pallas as pl
from jax.experimental.pallas import tpu as pltpu
from jax.experimental.pallas import tpu_sc as plsc

N = 10000
E = 160000
D_IN = 256
D_H = 512
G = 16

LC = 128
NCH = D_H // LC
NPAIR = NCH // 2
NCORE = 2
NTILE = 16
CPC = NCH // NCORE

EG = 128
GPT = 79
EPT = GPT * EG
E_PAD = NTILE * EPT

NP = 10112
ROWS_PT = NP // NTILE

HIST_W = 16
NH = 10240
HROWS_PT = NH // NTILE

BR = 2000
NR = N // BR

_mesh = plsc.VectorSubcoreMesh(core_axis_name="c", subcore_axis_name="s")


@functools.partial(
    pl.kernel,
    mesh=_mesh,
    out_type=jax.ShapeDtypeStruct((NCORE, NH, HIST_W), jnp.float32),
    scratch_types=[
        pltpu.VMEM((GPT, EG), jnp.int32),
        pltpu.VMEM((EG, HIST_W), jnp.float32),
        pltpu.VMEM_SHARED((NH, HIST_W), jnp.float32),
    ],
)
def _deg_kernel(dst_hbm, deg_hbm, dstv, ones_rows, hist_sh):
    c = lax.axis_index("c")
    s = lax.axis_index("s")
    pltpu.sync_copy(dst_hbm.at[s], dstv)
    lane = lax.iota(jnp.int32, 16)
    one_hot = jnp.where(lane == 0, 1.0, 0.0).astype(jnp.float32)
    zero = jnp.zeros((16,), jnp.float32)

    def _fillz(i, _):
        ones_rows[i] = zero
        return 0

    lax.fori_loop(0, EG, _fillz, 0)
    for z in range(HROWS_PT // EG):
        pltpu.sync_copy(ones_rows, hist_sh.at[pl.ds(s * HROWS_PT + z * EG, EG)])
    plsc.subcore_barrier()

    def _fill1(i, _):
        ones_rows[i] = one_hot
        return 0

    lax.fori_loop(0, EG, _fill1, 0)

    def _acc(g, _):
        pltpu.sync_copy(ones_rows, hist_sh.at[dstv.at[g]], add=True)
        return 0

    half = (GPT + 1) // 2

    def _acc_guard(p, _):
        g = c * half + p

        @pl.when(g < GPT)
        def _():
            _acc(g, 0)

        return 0

    lax.fori_loop(0, half, _acc_guard, 0)
    plsc.subcore_barrier()
    span = HROWS_PT
    pltpu.sync_copy(hist_sh.at[pl.ds(s * span, span)],
                    deg_hbm.at[c].at[pl.ds(s * span, span)])


@functools.partial(
    pl.kernel,
    mesh=_mesh,
    out_type=jax.ShapeDtypeStruct((NCH, NP, LC), jnp.float32),
    scratch_types=[
        pltpu.VMEM((GPT, EG), jnp.int32),
        pltpu.VMEM((1, EG), jnp.int32),
        pltpu.VMEM((1, EG), jnp.int32),
        pltpu.VMEM((EG, LC), jnp.float32),
        pltpu.VMEM((EG, LC), jnp.float32),
        pltpu.VMEM_SHARED((NP, LC), jnp.float32),
        pltpu.SemaphoreType.DMA,
        pltpu.SemaphoreType.DMA,
        pltpu.SemaphoreType.DMA,
        pltpu.SemaphoreType.DMA,
    ],
)
def _edge_scatter_kernel(h_hbm, src_hbm, dst_hbm, acc_hbm,
                         srcv, dst0, dst1, rows0, rows1, accsh,
                         sem0, sem1, semd0, semd1):
    c = lax.axis_index("c")
    s = lax.axis_index("s")
    pltpu.sync_copy(src_hbm.at[s], srcv)
    dsts = dst_hbm.at[s]
    zero = jnp.zeros((16,), jnp.float32)

    for ci in range(CPC):
        ch = c * CPC + ci
        hsrc = h_hbm.at[ch]

        def _zb(i, _):
            for k in range(LC // 16):
                rows0[i, pl.ds(k * 16, 16)] = zero
            return 0

        lax.fori_loop(0, EG, _zb, 0)
        nfull = ROWS_PT // EG
        for z in range(nfull):
            pltpu.sync_copy(rows0, accsh.at[pl.ds(s * ROWS_PT + z * EG, EG)])
        rem = ROWS_PT - nfull * EG
        if rem:
            pltpu.sync_copy(rows0.at[pl.ds(0, rem)],
                            accsh.at[pl.ds(s * ROWS_PT + nfull * EG, rem)])
        plsc.subcore_barrier()

        def _gather(g, buf, sem):
            return pltpu.make_async_copy(hsrc.at[srcv.at[g]], buf, sem)

        def _dstld(g, buf, sem):
            return pltpu.make_async_copy(dsts.at[pl.ds(g, 1)], buf, sem)

        _gather(0, rows0, sem0).start()
        _dstld(0, dst0, semd0).start()

        def _body(p, _):
            g0 = 2 * p
            g1 = g0 + 1
            _gather(g1, rows1, sem1).start()
            _dstld(g1, dst1, semd1).start()
            _gather(g0, rows0, sem0).wait()
            _dstld(g0, dst0, semd0).wait()
            pltpu.sync_copy(rows0, accsh.at[dst0.at[0]], add=True)
            _gather(g0 + 2, rows0, sem0).start()
            _dstld(g0 + 2, dst0, semd0).start()
            _gather(g1, rows1, sem1).wait()
            _dstld(g1, dst1, semd1).wait()
            pltpu.sync_copy(rows1, accsh.at[dst1.at[0]], add=True)
            return 0

        lax.fori_loop(0, (GPT - 1) // 2, _body, 0)
        _gather(GPT - 1, rows0, sem0).wait()
        _dstld(GPT - 1, dst0, semd0).wait()
        pltpu.sync_copy(rows0, accsh.at[dst0.at[0]], add=True)
        plsc.subcore_barrier()
        pltpu.sync_copy(accsh.at[pl.ds(s * ROWS_PT, ROWS_PT)],
                        acc_hbm.at[ch].at[pl.ds(s * ROWS_PT, ROWS_PT)])


def _mm1_body(x_ref, w_ref, deg_ref, out_ref):
    h = jnp.dot(x_ref[...], w_ref[...], preferred_element_type=jnp.float32)
    dinv = lax.rsqrt(deg_ref[0] + deg_ref[1] + 1.0)
    hs = h * dinv
    for cc in range(NCH):
        out_ref[cc] = hs[:, cc * LC:(cc + 1) * LC]


def _mm1(x, W1, degf):
    return pl.pallas_call(
        _mm1_body,
        grid=(NR,),
        in_specs=[
            pl.BlockSpec((BR, D_IN), lambda i: (i, 0)),
            pl.BlockSpec((D_IN, D_H), lambda i: (0, 0)),
            pl.BlockSpec((NCORE, BR, 1), lambda i: (0, i, 0)),
        ],
        out_specs=pl.BlockSpec((NCH, BR, LC), lambda i: (0, i, 0)),
        out_shape=jax.ShapeDtypeStruct((NCH, N, LC), jnp.float32),
    )(x, W1, degf)


def _mm2_body(acc_ref, h_ref, deg_ref, b_ref, w_ref, out_ref, accum):
    k = pl.program_id(1)
    dinv = lax.rsqrt(deg_ref[0] + deg_ref[1] + 1.0)
    acc = jnp.concatenate([acc_ref[0], acc_ref[1]], axis=1)
    h = jnp.concatenate([h_ref[0], h_ref[1]], axis=1)
    y = jnp.maximum((acc + h) * dinv + b_ref[0], 0.0)
    part = jnp.dot(y, w_ref[...], preferred_element_type=jnp.float32)

    @pl.when(k == 0)
    def _():
        accum[...] = jnp.zeros_like(accum)

    accum[...] += part

    @pl.when(k == NPAIR - 1)
    def _():
        h2 = accum[...] * dinv
        for cc in range(NCH):
            out_ref[cc] = h2[:, cc * LC:(cc + 1) * LC]


def _mm2(acc1, h1s, degf, b1r, W2):
    return pl.pallas_call(
        _mm2_body,
        grid=(NR, NPAIR),
        in_specs=[
            pl.BlockSpec((2, BR, LC), lambda i, k: (k, i, 0)),
            pl.BlockSpec((2, BR, LC), lambda i, k: (k, i, 0)),
            pl.BlockSpec((NCORE, BR, 1), lambda i, k: (0, i, 0)),
            pl.BlockSpec((1, 1, 2 * LC), lambda i, k: (k, 0, 0)),
            pl.BlockSpec((2 * LC, D_H), lambda i, k: (k, 0)),
        ],
        out_specs=pl.BlockSpec((NCH, BR, LC), lambda i, k: (0, i, 0)),
        out_shape=jax.ShapeDtypeStruct((NCH, N, LC), jnp.float32),
        scratch_shapes=[pltpu.VMEM((BR, D_H), jnp.float32)],
    )(acc1, h1s, degf, b1r, W2)


def _pool_body(bb_ref, acc_ref, h_ref, deg_ref, b_ref, batch_ref,
               w1_ref, b1_ref, w2_ref, b2_ref, out_ref, pooled):
    j = pl.program_id(0)
    i = pl.program_id(1)
    dinv = lax.rsqrt(deg_ref[0] + deg_ref[1] + 1.0)
    acc = jnp.concatenate([acc_ref[0], acc_ref[1]], axis=1)
    h = jnp.concatenate([h_ref[0], h_ref[1]], axis=1)
    y = jnp.maximum((acc + h) * dinv + b_ref[0], 0.0)
    bidx = batch_ref[...]
    neg = jnp.float32(-jnp.inf)

    @pl.when((i == 0) & (j == 0))
    def _():
        pooled[...] = jnp.full((NPAIR, G, 2 * LC), neg, jnp.float32)

    g_lo = bb_ref[i, 0]
    g_hi = bb_ref[i, 1]
    for jj in range(NPAIR):
        for g in range(G):
            @pl.when((j == jj) & (g_lo <= g) & (g <= g_hi))
            def _():
                v = jnp.max(jnp.where(bidx == g, y, neg), axis=0)
                pooled[jj, g, :] = jnp.maximum(pooled[jj, g, :], v)

    @pl.when((i == NR - 1) & (j == NPAIR - 1))
    def _():
        p = jnp.concatenate([pooled[0], pooled[1]], axis=1)
        hid = jnp.maximum(
            jnp.dot(p, w1_ref[...], preferred_element_type=jnp.float32)
            + b1_ref[...], 0.0)
        out_ref[...] = jnp.maximum(
            jnp.dot(hid, w2_ref[...], preferred_element_type=jnp.float32)
            + b2_ref[...], 0.0)


def _pool_mlp(bb, acc2, h2s, degf, b2r, batch2d, Wf1, bf1r, Wf2, bf2r):
    return pl.pallas_call(
        _pool_body,
        grid=(NPAIR, NR),
        in_specs=[
            pl.BlockSpec(memory_space=pltpu.SMEM),
            pl.BlockSpec((2, BR, LC), lambda j, i: (j, i, 0)),
            pl.BlockSpec((2, BR, LC), lambda j, i: (j, i, 0)),
            pl.BlockSpec((NCORE, BR, 1), lambda j, i: (0, i, 0)),
            pl.BlockSpec((1, 1, 2 * LC), lambda j, i: (j, 0, 0)),
            pl.BlockSpec((BR, 1), lambda j, i: (i, 0)),
            pl.BlockSpec((D_H, 64), lambda j, i: (0, 0)),
            pl.BlockSpec((1, 64), lambda j, i: (0, 0)),
            pl.BlockSpec((64, 32), lambda j, i: (0, 0)),
            pl.BlockSpec((1, 32), lambda j, i: (0, 0)),
        ],
        out_specs=pl.BlockSpec((G, 32), lambda j, i: (0, 0)),
        out_shape=jax.ShapeDtypeStruct((G, 32), jnp.float32),
        scratch_shapes=[pltpu.VMEM((NPAIR, G, 2 * LC), jnp.float32)],
    )(bb, acc2, h2s, degf, b2r, batch2d, Wf1, bf1r, Wf2, bf2r)


def kernel(x, edge_index, batch, num_graphs, ptr, W1, b1, W2, b2,
           Wf1, bf1, Wf2, bf2):
    pad = E_PAD - E
    pad_src = (jnp.arange(pad, dtype=jnp.int32) * 53) % N
    pad_dst = N + jnp.arange(pad, dtype=jnp.int32) % (NP - N)
    src2 = jnp.concatenate([edge_index[0], pad_src]).reshape(NTILE, GPT, EG)
    dst2 = jnp.concatenate([edge_index[1], pad_dst]).reshape(NTILE, GPT, EG)

    degh = _deg_kernel(dst2)
    degf = degh[:, :N, 0:1]

    h1s = _mm1(x, W1, degf)
    acc1 = _edge_scatter_kernel(h1s, src2, dst2)
    h2s = _mm2(acc1, h1s, degf, b1.reshape(NPAIR, 1, 2 * LC), W2)
    acc2 = _edge_scatter_kernel(h2s, src2, dst2)
    bb = batch.reshape(NR, BR)[:, jnp.array([0, BR - 1])]
    return _pool_mlp(bb, acc2, h2s, degf, b2.reshape(NPAIR, 1, 2 * LC),
                     batch.reshape(N, 1), Wf1, bf1.reshape(1, 64),
                     Wf2, bf2.reshape(1, 32))
